# baseline (speedup 1.0000x reference)
import functools

import jax
import jax.numpy as jnp
from jax import lax
from jax.experimental import pallas as pl
from jax.experimental.pallas import tpu as pltpu
from jax.experimental.pallas import tpu_sc as plsc

_DEPTH = 1000
_BATCH = 4096
_FIELDS = 26
_ROWS = _BATCH * _FIELDS
_TOT = _ROWS * _DEPTH
_NW = 32
_RPW = _ROWS // _NW
_L = 16
_IPD = 128
_ND = _RPW // _IPD

_FR = 832
_FC = 128000
_FBR = 8


def _fill_body(off_ref, o_ref):
    o_ref[...] = jnp.full((_FBR, _FC), off_ref[0, 0], jnp.float32)


_tc_fill = pl.pallas_call(
    _fill_body,
    grid=(_FR // _FBR,),
    in_specs=[pl.BlockSpec(memory_space=pltpu.SMEM)],
    out_specs=pl.BlockSpec((_FBR, _FC), lambda i: (i, 0)),
    out_shape=jax.ShapeDtypeStruct((_FR, _FC), jnp.float32),
)


def _scat_body(idx_hbm, on_hbm, out_hbm, idx_v, offs_v, vals_v, on_v, sem):
    wid = lax.axis_index("s") * 2 + lax.axis_index("c")
    row0 = wid * _RPW

    pltpu.sync_copy(idx_hbm.at[pl.ds(row0 * 1, _RPW)], idx_v)
    pltpu.sync_copy(on_hbm, on_v)
    on_vec = on_v[...]
    lane = lax.iota(jnp.int32, _L)

    for u in range(_IPD // _L):
        vals_v[pl.ds(u * _L, _L)] = on_vec

    for k in range(_RPW // _L):
        idx16 = idx_v[pl.ds(k * _L, _L)]
        offs = idx16 + (row0 + k * _L + lane) * _DEPTH
        offs_v[k // (_IPD // _L), pl.ds((k % (_IPD // _L)) * _L, _L)] = offs

    for j in range(_ND):
        pltpu.make_async_copy(vals_v, out_hbm.at[offs_v.at[j]], sem).start()
    for j in range(_ND):
        pltpu.make_async_copy(vals_v, out_hbm.at[offs_v.at[j]], sem).wait()


_sc_scatter = functools.partial(
    pl.kernel,
    out_type=(),
    mesh=plsc.VectorSubcoreMesh(core_axis_name="c", subcore_axis_name="s"),
    compiler_params=pltpu.CompilerParams(needs_layout_passes=False),
    scratch_types=[
        pltpu.VMEM((_RPW,), jnp.int32),
        pltpu.VMEM((_ND, _IPD), jnp.int32),
        pltpu.VMEM((_IPD,), jnp.float32),
        pltpu.VMEM((_L,), jnp.float32),
        pltpu.SemaphoreType.DMA,
    ],
)(_scat_body)


def kernel(inputs, on_value, off_value):
    idx = inputs.reshape(_ROWS)
    on16 = jnp.broadcast_to(on_value.astype(jnp.float32), (_L,))
    off11 = off_value.astype(jnp.float32).reshape(1, 1)
    filled = _tc_fill(off11).reshape(_TOT)
    ref = jax.new_ref(filled)
    _sc_scatter(idx, on16, ref)
    out = jax.freeze(ref)
    return out.reshape(_BATCH, _FIELDS, _DEPTH)


# speedup vs baseline: 12.0768x; 12.0768x over previous
import jax
import jax.numpy as jnp
from jax import lax
from jax.experimental import pallas as pl
from jax.experimental.pallas import tpu as pltpu

_DEPTH = 1000
_BATCH = 4096
_FIELDS = 26
_FBD = 200


def _oh_body(idx_ref, on_ref, off_ref, o_ref):
    dc = pl.program_id(1)
    idx_b = idx_ref[...]
    dd = lax.broadcasted_iota(jnp.int32, (1, _FBD, _BATCH), 1) + dc * _FBD
    o_ref[...] = jnp.where(dd == idx_b, on_ref[0, 0], off_ref[0, 0])


_tc_onehot = pl.pallas_call(
    _oh_body,
    grid=(_FIELDS, _DEPTH // _FBD),
    in_specs=[
        pl.BlockSpec((1, 1, _BATCH), lambda f, dc: (f, 0, 0)),
        pl.BlockSpec(memory_space=pltpu.SMEM),
        pl.BlockSpec(memory_space=pltpu.SMEM),
    ],
    out_specs=pl.BlockSpec((1, _FBD, _BATCH), lambda f, dc: (f, dc, 0)),
    out_shape=jax.ShapeDtypeStruct((_FIELDS, _DEPTH, _BATCH), jnp.float32),
)


def kernel(inputs, on_value, off_value):
    idx_t = jnp.transpose(inputs).reshape(_FIELDS, 1, _BATCH)
    on11 = on_value.astype(jnp.float32).reshape(1, 1)
    off11 = off_value.astype(jnp.float32).reshape(1, 1)
    out = _tc_onehot(idx_t, on11, off11)
    return jnp.transpose(out, (2, 0, 1))


# speedup vs baseline: 12.4280x; 1.0291x over previous
import jax
import jax.numpy as jnp
from jax import lax
from jax.experimental import pallas as pl
from jax.experimental.pallas import tpu as pltpu

_DEPTH = 1000
_BATCH = 4096
_FIELDS = 26
_FBD = 1000


def _oh_body(idx_ref, on_ref, off_ref, o_ref):
    dc = pl.program_id(1)
    idx_b = idx_ref[...]
    dd = lax.broadcasted_iota(jnp.int32, (1, _FBD, _BATCH), 1) + dc * _FBD
    o_ref[...] = jnp.where(dd == idx_b, on_ref[0, 0], off_ref[0, 0])


_tc_onehot = pl.pallas_call(
    _oh_body,
    grid=(_FIELDS, _DEPTH // _FBD),
    in_specs=[
        pl.BlockSpec((1, 1, _BATCH), lambda f, dc: (f, 0, 0)),
        pl.BlockSpec(memory_space=pltpu.SMEM),
        pl.BlockSpec(memory_space=pltpu.SMEM),
    ],
    out_specs=pl.BlockSpec((1, _FBD, _BATCH), lambda f, dc: (f, dc, 0)),
    out_shape=jax.ShapeDtypeStruct((_FIELDS, _DEPTH, _BATCH), jnp.float32),
)


def kernel(inputs, on_value, off_value):
    idx_t = jnp.transpose(inputs).reshape(_FIELDS, 1, _BATCH)
    on11 = on_value.astype(jnp.float32).reshape(1, 1)
    off11 = off_value.astype(jnp.float32).reshape(1, 1)
    out = _tc_onehot(idx_t, on11, off11)
    return jnp.transpose(out, (2, 0, 1))
